# baseline (speedup 1.0000x reference)
import functools

import jax
import jax.numpy as jnp
from jax import lax
from jax.experimental import pallas as pl
from jax.experimental.pallas import tpu as pltpu
from jax.experimental.pallas import tpu_sc as plsc

N_TOKEN = 1000000
EMB_DIM = 64
BATCH = 16384
HIST = 20

NC = 2
NS = 16
NW = NC * NS

B = BATCH * HIST
B_PER_W = B // NW
CHUNK = 128
N_CHUNKS = B_PER_W // CHUNK
NBUF = 4
T = N_CHUNKS // NBUF


def _emb_body(idx_hbm, table_hbm, out_hbm, idx_v, rows_v, sems):
    wid = lax.axis_index("s") * NC + lax.axis_index("c")
    base = wid * B_PER_W

    pltpu.sync_copy(idx_hbm.at[wid], idx_v)

    def start(b, j):
        pltpu.async_copy(table_hbm.at[idx_v.at[j]], rows_v.at[b], sems.at[b])

    def finish(b, j):
        pltpu.make_async_copy(
            table_hbm.at[idx_v.at[j]], rows_v.at[b], sems.at[b]
        ).wait()
        pltpu.sync_copy(rows_v.at[b], out_hbm.at[pl.ds(base + j * CHUNK, CHUNK)])

    for b in range(NBUF):
        start(b, b)

    def body(it, carry):
        for b in range(NBUF):
            j = it * NBUF + b
            finish(b, j)
            start(b, j + NBUF)
        return carry

    lax.fori_loop(0, T - 1, body, 0)

    for b in range(NBUF):
        finish(b, (T - 1) * NBUF + b)


@jax.jit
def _emb_call(x_flat, weight):
    mesh = plsc.VectorSubcoreMesh(core_axis_name="c", subcore_axis_name="s")
    kern = pl.kernel(
        _emb_body,
        out_type=jax.ShapeDtypeStruct((B, EMB_DIM), jnp.float32),
        mesh=mesh,
        scratch_types=[
            pltpu.VMEM((N_CHUNKS, CHUNK), jnp.int32),
            pltpu.VMEM((NBUF, CHUNK, EMB_DIM), jnp.float32),
            pltpu.SemaphoreType.DMA((NBUF,)),
        ],
        compiler_params=pltpu.CompilerParams(use_tc_tiling_on_sc=False),
    )
    return kern(x_flat, weight)


def kernel(x, weight):
    x_flat = x.reshape(NW, N_CHUNKS, CHUNK).astype(jnp.int32)
    out = _emb_call(x_flat, weight)
    return out.reshape(BATCH, HIST, EMB_DIM)
